# baseline (speedup 1.0000x reference)
import functools

import jax
import jax.numpy as jnp
from jax import lax
from jax.experimental import pallas as pl
from jax.experimental.pallas import tpu as pltpu
from jax.experimental.pallas import tpu_sc as plsc

BATCH = 16384
DIM = 64
NUM_CORES = 2
NUM_SUBCORES = 16
NUM_WORKERS = NUM_CORES * NUM_SUBCORES
ROWS_PER_WORKER = BATCH // NUM_WORKERS
CHUNK = 128
NCHUNK = ROWS_PER_WORKER // CHUNK


def _body(sub_hbm, idx_hbm, obj_hbm, table_hbm,
          sub_out, rel_out, obj_out,
          idx_v, rows_v, sem, copy_sem):
    wid = lax.axis_index("s") * NUM_CORES + lax.axis_index("c")
    base = wid * ROWS_PER_WORKER
    sl = pl.ds(base, ROWS_PER_WORKER)
    sub_copy = pltpu.async_copy(sub_hbm.at[sl], sub_out.at[sl], copy_sem)
    obj_copy = pltpu.async_copy(obj_hbm.at[sl], obj_out.at[sl], copy_sem)
    pltpu.sync_copy(idx_hbm.at[pl.ds(wid * NCHUNK, NCHUNK)], idx_v)
    gathers = [
        pltpu.async_copy(
            table_hbm.at[idx_v.at[j]],
            rows_v.at[pl.ds(j * CHUNK, CHUNK)],
            sem,
        )
        for j in range(NCHUNK)
    ]
    for g in gathers:
        g.wait()
    pltpu.sync_copy(rows_v, rel_out.at[sl])
    sub_copy.wait()
    obj_copy.wait()


_fused = functools.partial(
    pl.kernel,
    out_type=(
        jax.ShapeDtypeStruct((BATCH, DIM), jnp.float32),
        jax.ShapeDtypeStruct((BATCH, DIM), jnp.float32),
        jax.ShapeDtypeStruct((BATCH, DIM), jnp.float32),
    ),
    mesh=plsc.VectorSubcoreMesh(core_axis_name="c", subcore_axis_name="s"),
    scratch_types=[
        pltpu.VMEM((NCHUNK, CHUNK), jnp.int32),
        pltpu.VMEM((ROWS_PER_WORKER, DIM), jnp.float32),
        pltpu.SemaphoreType.DMA,
        pltpu.SemaphoreType.DMA,
    ],
    compiler_params=pltpu.CompilerParams(use_tc_tiling_on_sc=False),
)(_body)


def kernel(subject_embeddings, relation_ids, object_embeddings, relation_table):
    idx2d = relation_ids.astype(jnp.int32).reshape(NUM_WORKERS * NCHUNK, CHUNK)
    return _fused(subject_embeddings, idx2d, object_embeddings, relation_table)


# speedup vs baseline: 3.5921x; 3.5921x over previous
import functools

import jax
import jax.numpy as jnp
from jax import lax
from jax.experimental import pallas as pl
from jax.experimental.pallas import tpu as pltpu
from jax.experimental.pallas import tpu_sc as plsc

BATCH = 16384
DIM = 64
NUM_CORES = 2
NUM_SUBCORES = 16
NUM_WORKERS = NUM_CORES * NUM_SUBCORES
ROWS_PER_WORKER = BATCH // NUM_WORKERS
CHUNK = 128
NCHUNK = ROWS_PER_WORKER // CHUNK


def _body(sub_hbm, idx_hbm, obj_hbm, table_hbm,
          sub_out, rel_out, obj_out,
          idx_v, rows_v, sub_v, obj_v,
          sem_sub, sem_obj, sem_g, sem_out):
    wid = lax.axis_index("s") * NUM_CORES + lax.axis_index("c")
    base = wid * ROWS_PER_WORKER
    sl = pl.ds(base, ROWS_PER_WORKER)
    sub_in = pltpu.async_copy(sub_hbm.at[sl], sub_v, sem_sub)
    obj_in = pltpu.async_copy(obj_hbm.at[sl], obj_v, sem_obj)
    pltpu.sync_copy(idx_hbm.at[pl.ds(wid * NCHUNK, NCHUNK)], idx_v)
    gathers = [
        pltpu.async_copy(
            table_hbm.at[idx_v.at[j]],
            rows_v.at[pl.ds(j * CHUNK, CHUNK)],
            sem_g,
        )
        for j in range(NCHUNK)
    ]
    sub_in.wait()
    sub_out_c = pltpu.async_copy(sub_v, sub_out.at[sl], sem_out)
    obj_in.wait()
    obj_out_c = pltpu.async_copy(obj_v, obj_out.at[sl], sem_out)
    for g in gathers:
        g.wait()
    rel_out_c = pltpu.async_copy(rows_v, rel_out.at[sl], sem_out)
    sub_out_c.wait()
    obj_out_c.wait()
    rel_out_c.wait()


_fused = functools.partial(
    pl.kernel,
    out_type=(
        jax.ShapeDtypeStruct((BATCH, DIM), jnp.float32),
        jax.ShapeDtypeStruct((BATCH, DIM), jnp.float32),
        jax.ShapeDtypeStruct((BATCH, DIM), jnp.float32),
    ),
    mesh=plsc.VectorSubcoreMesh(core_axis_name="c", subcore_axis_name="s"),
    scratch_types=[
        pltpu.VMEM((NCHUNK, CHUNK), jnp.int32),
        pltpu.VMEM((ROWS_PER_WORKER, DIM), jnp.float32),
        pltpu.VMEM((ROWS_PER_WORKER, DIM), jnp.float32),
        pltpu.VMEM((ROWS_PER_WORKER, DIM), jnp.float32),
        pltpu.SemaphoreType.DMA,
        pltpu.SemaphoreType.DMA,
        pltpu.SemaphoreType.DMA,
        pltpu.SemaphoreType.DMA,
    ],
    compiler_params=pltpu.CompilerParams(use_tc_tiling_on_sc=False),
)(_body)


def kernel(subject_embeddings, relation_ids, object_embeddings, relation_table):
    idx2d = relation_ids.astype(jnp.int32).reshape(NUM_WORKERS * NCHUNK, CHUNK)
    return _fused(subject_embeddings, idx2d, object_embeddings, relation_table)


# speedup vs baseline: 7.1971x; 2.0036x over previous
import functools

import jax
import jax.numpy as jnp
from jax import lax
from jax.experimental import pallas as pl
from jax.experimental.pallas import tpu as pltpu
from jax.experimental.pallas import tpu_sc as plsc

BATCH = 16384
DIM = 64
NUM_CORES = 2
NUM_SUBCORES = 16
NUM_WORKERS = NUM_CORES * NUM_SUBCORES
ROWS_PER_WORKER = BATCH // NUM_WORKERS
CHUNK = 128
NCHUNK = ROWS_PER_WORKER // CHUNK


def _gather_body(table_hbm, idx_hbm, out_hbm, idx_v, rows_v, sem):
    wid = lax.axis_index("s") * NUM_CORES + lax.axis_index("c")
    base = wid * ROWS_PER_WORKER
    pltpu.sync_copy(idx_hbm.at[pl.ds(wid * NCHUNK, NCHUNK)], idx_v)
    gathers = [
        pltpu.async_copy(
            table_hbm.at[idx_v.at[j]],
            rows_v.at[pl.ds(j * CHUNK, CHUNK)],
            sem,
        )
        for j in range(NCHUNK)
    ]
    for g in gathers:
        g.wait()
    pltpu.sync_copy(rows_v, out_hbm.at[pl.ds(base, ROWS_PER_WORKER)])


_gather = functools.partial(
    pl.kernel,
    out_type=jax.ShapeDtypeStruct((BATCH, DIM), jnp.float32),
    mesh=plsc.VectorSubcoreMesh(core_axis_name="c", subcore_axis_name="s"),
    scratch_types=[
        pltpu.VMEM((NCHUNK, CHUNK), jnp.int32),
        pltpu.VMEM((ROWS_PER_WORKER, DIM), jnp.float32),
        pltpu.SemaphoreType.DMA,
    ],
    compiler_params=pltpu.CompilerParams(use_tc_tiling_on_sc=False),
    cost_estimate=pl.CostEstimate(
        flops=0,
        transcendentals=0,
        bytes_accessed=2 * BATCH * DIM * 4 + BATCH * 4,
    ),
)(_gather_body)


def kernel(subject_embeddings, relation_ids, object_embeddings, relation_table):
    idx2d = relation_ids.astype(jnp.int32).reshape(NUM_WORKERS * NCHUNK, CHUNK)
    relation_embeddings = _gather(relation_table, idx2d)
    return (subject_embeddings, relation_embeddings, object_embeddings)
